# Initial kernel scaffold; baseline (speedup 1.0000x reference)
import jax
import jax.numpy as jnp
from jax.experimental import pallas as pl


def kernel(x, vectors):
    raise NotImplementedError("write your pallas kernel here")


# speedup vs baseline: 1.9565x; 1.9565x over previous
import functools

import jax
import jax.numpy as jnp
from jax import lax
from jax.experimental import pallas as pl
from jax.experimental.pallas import tpu as pltpu
from jax.experimental.pallas import tpu_sc as plsc

_HU = (-1000.0, -900.0, -400.0, -100.0, -50.0, -10.0,
       20.0, 40.0, 60.0, 100.0, 800.0, 1000.0)
_NC = 2
_NS = 16
_NW = _NC * _NS
_NPAIR = 128
_PAIRS_PER_W = _NPAIR // _NW
_PLANE = 24 * 24 * 24
_LANES = 16
_STEPS = _PLANE // _LANES
_VD = 8
_VPAD = 16


def _tec_body(x_hbm, vt_hbm, out_hbm, x_v, vt_v, out_v):
    wid = lax.axis_index("s") * _NC + lax.axis_index("c")
    pltpu.sync_copy(vt_hbm, vt_v)

    cols = [vt_v[pl.ds(c * _VPAD, _LANES)] for c in range(_VD)]

    def do_pair(pp, carry):
        pair = wid * _PAIRS_PER_W + pp
        pltpu.sync_copy(x_hbm.at[pair], x_v)

        def step(i, carry2):
            off = i * _LANES
            xv = x_v[pl.ds(off, _LANES)]
            bin_ = jnp.where(xv >= _HU[0], 1, 0)
            for t in _HU[1:]:
                bin_ = bin_ + jnp.where(xv >= t, 1, 0)
            for c in range(_VD):
                val = jnp.take_along_axis(cols[c], bin_, axis=0)
                out_v[c, pl.ds(off, _LANES)] = val
            return carry2

        lax.fori_loop(0, _STEPS, step, 0, unroll=2)
        for c in range(_VD):
            pltpu.sync_copy(out_v.at[c], out_hbm.at[pair * _VD + c])
        return carry

    lax.fori_loop(0, _PAIRS_PER_W, do_pair, 0)


def _make_sc_call():
    mesh = plsc.VectorSubcoreMesh(core_axis_name="c", subcore_axis_name="s",
                                  num_cores=_NC, num_subcores=_NS)
    return pl.kernel(
        _tec_body,
        out_type=jax.ShapeDtypeStruct((_NPAIR * _VD, _PLANE), jnp.float32),
        mesh=mesh,
        scratch_types=[
            pltpu.VMEM((_PLANE,), jnp.float32),
            pltpu.VMEM((_VD * _VPAD,), jnp.float32),
            pltpu.VMEM((_VD, _PLANE), jnp.float32),
        ],
    )


@jax.jit
def kernel(x, vectors):
    B, C, D, H, W = x.shape
    P = 4
    Dp, Hp, Wp = D // P, H // P, W // P
    x_t = x[:, 0].reshape(B, Dp, P, Hp, P, Wp, P)
    x_t = x_t.transpose(0, 2, 4, 6, 1, 3, 5).reshape(_NPAIR, _PLANE)
    vt = jnp.zeros((_VD, _VPAD), jnp.float32).at[:, : vectors.shape[0]].set(vectors.T)
    vt = vt.reshape(_VD * _VPAD)
    out = _make_sc_call()(x_t, vt)
    return out.reshape(B, P * P * P * _VD, Dp, Hp, Wp)


# speedup vs baseline: 4.3737x; 2.2355x over previous
import jax
import jax.numpy as jnp
from jax import lax
from jax.experimental import pallas as pl
from jax.experimental.pallas import tpu as pltpu
from jax.experimental.pallas import tpu_sc as plsc

_HU = (-1000.0, -900.0, -400.0, -100.0, -50.0, -10.0,
       20.0, 40.0, 60.0, 100.0, 800.0, 1000.0)
_NC = 2
_NS = 16
_L = 16
_P = 4
_G = 24
_W = 96
_VD = 8
_VPAD = 16
_PLANE = _G * _G * _G
_CH = _P * _P * _P * _VD
_NROW = 1024
_CHUNK = _G * _W
_SEG = _G * _G


def _bin16(xv):
    b = jnp.where(xv >= _HU[0], 1, 0)
    for t in _HU[1:]:
        b = b + jnp.where(xv >= t, 1, 0)
    return b


def _tec_body(x_hbm, vt_hbm, out_hbm, xc_v, vt_v, bins_v, out_v):
    wid = lax.axis_index("s") * _NC + lax.axis_index("c")
    b = wid // 16
    pd = (wid % 16) // 4
    ph = wid % 4
    base32 = wid * 32

    pltpu.sync_copy(vt_hbm, vt_v)
    cols = [vt_v[pl.ds(c * _VPAD, _L)] for c in range(_VD)]

    lane = lax.iota(jnp.int32, _L)
    perm_pw = [(4 * lane + pw) % _L for pw in range(_P)]
    sel_m = [lane < 4, lane < 8, lane < 12]

    def _merge4(za, zb, zc, zd):
        return jnp.where(sel_m[0], za,
                         jnp.where(sel_m[1], zb,
                                   jnp.where(sel_m[2], zc, zd)))

    def do_chunk(i, carry):
        pltpu.sync_copy(x_hbm.at[b, pl.ds(2 * i, 2), pd, :, ph, :], xc_v)

        def do_rowpair(hp2, c2):
            for dd in range(2):
                bv = []
                for rr in range(2):
                    bv += [_bin16(xc_v[dd, 2 * hp2 + rr, pl.ds(k * _L, _L)])
                           for k in range(6)]
                off = dd * _SEG + hp2 * 2 * _G
                for pw in range(_P):
                    z = [jnp.take_along_axis(v, perm_pw[pw], axis=0) for v in bv]
                    bins_v[pw, pl.ds(off, _L)] = _merge4(z[0], z[1], z[2], z[3])
                    bins_v[pw, pl.ds(off + _L, _L)] = _merge4(z[4], z[5], z[6], z[7])
                    bins_v[pw, pl.ds(off + 2 * _L, _L)] = _merge4(z[8], z[9], z[10], z[11])
            return c2

        lax.fori_loop(0, _G // 2, do_rowpair, 0)

        def do_gather(t, c3):
            for pw in range(_P):
                bv = bins_v[pw, pl.ds(t * _L, _L)]
                for c in range(_VD):
                    out_v[pw * _VD + c, pl.ds(t * _L, _L)] = (
                        jnp.take_along_axis(cols[c], bv, axis=0))
            return c3

        lax.fori_loop(0, 2 * _SEG // _L, do_gather, 0)
        pltpu.sync_copy(out_v, out_hbm.at[pl.ds(base32, 32),
                                          pl.ds(i * 2 * _SEG, 2 * _SEG)])
        return carry

    lax.fori_loop(0, _G // 2, do_chunk, 0)


def _make_sc_call():
    mesh = plsc.VectorSubcoreMesh(core_axis_name="c", subcore_axis_name="s",
                                  num_cores=_NC, num_subcores=_NS)
    return pl.kernel(
        _tec_body,
        out_type=jax.ShapeDtypeStruct((_NROW, _PLANE), jnp.float32),
        mesh=mesh,
        scratch_types=[
            pltpu.VMEM((2, _G, _W), jnp.float32),
            pltpu.VMEM((_VD * _VPAD,), jnp.float32),
            pltpu.VMEM((_P, 2 * _SEG), jnp.int32),
            pltpu.VMEM((32, 2 * _SEG), jnp.float32),
        ],
    )


@jax.jit
def kernel(x, vectors):
    B, C, D, H, W = x.shape
    x6 = x.reshape(B, D // _P, _P, H // _P, _P, W)
    vt = jnp.zeros((_VD, _VPAD), jnp.float32).at[:, : vectors.shape[0]].set(vectors.T)
    out = _make_sc_call()(x6, vt.reshape(_VD * _VPAD))
    return out.reshape(B, _CH, D // _P, H // _P, W // _P)


# speedup vs baseline: 5.7034x; 1.3040x over previous
import jax
import jax.numpy as jnp
from jax import lax
from jax.experimental import pallas as pl
from jax.experimental.pallas import tpu as pltpu
from jax.experimental.pallas import tpu_sc as plsc

_HU = (-1000.0, -900.0, -400.0, -100.0, -50.0, -10.0,
       20.0, 40.0, 60.0, 100.0, 800.0, 1000.0)
_NC = 2
_NS = 16
_L = 16
_P = 4
_G = 24
_W = 96
_VD = 8
_NV = 13
_CH = _P * _P * _P * _VD
_UNITS_PER_W = 2 * _G * _G // (_NC * _NS)


def _bin16(xv):
    b = jnp.where(xv >= _HU[0], 1, 0)
    for t in _HU[1:]:
        b = b + jnp.where(xv >= t, 1, 0)
    return b


def _tec_body(x_hbm, vtp_hbm, out_hbm, xb_v, vtp_v, bp_v, out_v):
    wid = lax.axis_index("s") * _NC + lax.axis_index("c")

    pltpu.sync_copy(vtp_hbm, vtp_v)

    lane = lax.iota(jnp.int32, _L)
    pat_e = (2 * lane) % _L
    pat_o = (2 * lane + 1) % _L
    low8 = lane < 8

    def do_unit(k, carry):
        u = wid * _UNITS_PER_W + k
        b = u // (_G * _G)
        r = u % (_G * _G)
        dp = r // _G
        hp = r % _G
        pltpu.sync_copy(x_hbm.at[b, dp, :, hp, :, :], xb_v)

        def do_row(pdph, c2):
            bv = [_bin16(xb_v[pdph // 4, pdph % 4, pl.ds(q * _L, _L)])
                  for q in range(6)]
            for m in range(3):
                b1, b2 = bv[2 * m], bv[2 * m + 1]
                ev = jnp.where(low8, jnp.take_along_axis(b1, pat_e, axis=0),
                               jnp.take_along_axis(b2, pat_e, axis=0))
                od = jnp.where(low8, jnp.take_along_axis(b1, pat_o, axis=0),
                               jnp.take_along_axis(b2, pat_o, axis=0))
                bp_v[pdph, pl.ds(m * _L, _L)] = ev * _L + od
            return c2

        lax.fori_loop(0, _P * _P, do_row, 0)

        def do_pdph(pdph, c3):
            off = pdph * 2 * _L
            for m in range(3):
                bpv = bp_v[pdph, pl.ds(m * _L, _L)]
                for l in range(_L):
                    p = m * _L + l
                    wp, j = p // 2, p % 2
                    out_v[wp, pl.ds(j * _L + off, _L)] = vtp_v[bpv[l]]
            return c3

        lax.fori_loop(0, _P * _P, do_pdph, 0)
        pltpu.sync_copy(out_v, out_hbm.at[b, dp, hp])
        return carry

    lax.fori_loop(0, _UNITS_PER_W, do_unit, 0)


def _make_sc_call():
    mesh = plsc.VectorSubcoreMesh(core_axis_name="c", subcore_axis_name="s",
                                  num_cores=_NC, num_subcores=_NS)
    return pl.kernel(
        _tec_body,
        out_type=jax.ShapeDtypeStruct((2, _G, _G, _G, _CH), jnp.float32),
        mesh=mesh,
        scratch_types=[
            pltpu.VMEM((_P, _P, _W), jnp.float32),
            pltpu.VMEM((_NV * _L, _L), jnp.float32),
            pltpu.VMEM((_P * _P, 2 * _G), jnp.int32),
            pltpu.VMEM((_G, _CH), jnp.float32),
        ],
    )


@jax.jit
def kernel(x, vectors):
    B, C, D, H, W = x.shape
    x6 = x.reshape(B, D // _P, _P, H // _P, _P, W)
    va = jnp.broadcast_to(vectors[:, None, :], (_NV, _NV, _VD))
    vb = jnp.broadcast_to(vectors[None, :, :], (_NV, _NV, _VD))
    vtp = jnp.concatenate([va, vb], axis=-1)
    vtp = jnp.pad(vtp, ((0, 0), (0, _L - _NV), (0, 0)))
    vtp = vtp.reshape(_NV * _L, _L)
    out = _make_sc_call()(x6, vtp)
    return jnp.transpose(out, (0, 4, 1, 2, 3))
